# Initial kernel scaffold; baseline (speedup 1.0000x reference)
import jax
import jax.numpy as jnp
from jax.experimental import pallas as pl


def kernel(seq_len, rel_pos_emb):
    raise NotImplementedError("write your pallas kernel here")


# speedup vs baseline: 13.0901x; 13.0901x over previous
import functools

import jax
import jax.numpy as jnp
from jax import lax
from jax.experimental import pallas as pl
from jax.experimental.pallas import tpu as pltpu
from jax.experimental.pallas import tpu_sc as plsc

_N = 1024
_D = 128
_NC = 2
_NS = 16
_NW = _NC * _NS
_ROWS_PER_W = _N // _NW
_STAGE = 2048 // _NS


def _rpe_body(tbl_hbm, out_hbm, idx_v, stage_v, shared, sem):
    c = lax.axis_index("c")
    s = lax.axis_index("s")

    k0 = s * _STAGE
    for t in range(_STAGE // 16):
        v = (2046 - k0 - 16 * t) - lax.iota(jnp.int32, 16)
        idx_v[pl.ds(16 * t, 16)] = jnp.maximum(v, 0)
    pltpu.async_copy(tbl_hbm.at[idx_v], stage_v, sem).wait()
    pltpu.sync_copy(stage_v, shared.at[pl.ds(k0, _STAGE)])
    plsc.subcore_barrier()

    w = s * _NC + c

    def body(j, carry):
        i = w * _ROWS_PER_W + j
        pltpu.sync_copy(shared.at[pl.ds(1023 - i, _N)], out_hbm.at[i])
        return carry

    lax.fori_loop(0, _ROWS_PER_W, body, 0)


_rpe = functools.partial(
    pl.kernel,
    out_type=jax.ShapeDtypeStruct((_N, _N, _D), jnp.float32),
    mesh=plsc.VectorSubcoreMesh(
        core_axis_name="c", subcore_axis_name="s",
        num_cores=_NC, num_subcores=_NS),
    scratch_types=[
        pltpu.VMEM((_STAGE,), jnp.int32),
        pltpu.VMEM((_STAGE, _D), jnp.float32),
        pltpu.VMEM_SHARED((2048, _D), jnp.float32),
        pltpu.SemaphoreType.DMA,
    ],
)(_rpe_body)


def kernel(seq_len, rel_pos_emb):
    del seq_len
    return _rpe(rel_pos_emb)


# speedup vs baseline: 20.6782x; 1.5797x over previous
import functools

import jax
import jax.numpy as jnp
from jax import lax
from jax.experimental import pallas as pl
from jax.experimental.pallas import tpu as pltpu
from jax.experimental.pallas import tpu_sc as plsc

_N = 1024
_D = 128
_NC = 2
_NS = 16
_NW = _NC * _NS
_ROWS_PER_W = _N // _NW
_STAGE = 2048 // _NS
_H = _N // 2
_WIN = _H + _ROWS_PER_W


def _rpe_body(tbl_hbm, out_hbm, idx_v, stage_v, win_v, shared, sem, sem2):
    c = lax.axis_index("c")
    s = lax.axis_index("s")
    w = s * _NC + c

    k0 = s * _STAGE
    for t in range(_STAGE // 16):
        v = (2046 - k0 - 16 * t) - lax.iota(jnp.int32, 16)
        idx_v[pl.ds(16 * t, 16)] = jnp.maximum(v, 0)
    pltpu.async_copy(tbl_hbm.at[idx_v], stage_v, sem).wait()
    pltpu.sync_copy(stage_v, shared.at[pl.ds(k0, _STAGE)])
    plsc.subcore_barrier()

    w0 = (_N - _ROWS_PER_W) - _ROWS_PER_W * w
    pltpu.sync_copy(shared.at[pl.ds(w0, _WIN)], win_v)

    def body(j, carry):
        i = w * _ROWS_PER_W + j
        cp = pltpu.async_copy(
            win_v.at[pl.ds(_ROWS_PER_W - 1 - j, _H)],
            out_hbm.at[i, pl.ds(0, _H)], sem2)
        pltpu.sync_copy(shared.at[pl.ds(1535 - i, _H)],
                        out_hbm.at[i, pl.ds(_H, _H)])
        cp.wait()
        return carry

    lax.fori_loop(0, _ROWS_PER_W, body, 0)


_rpe = functools.partial(
    pl.kernel,
    out_type=jax.ShapeDtypeStruct((_N, _N, _D), jnp.float32),
    mesh=plsc.VectorSubcoreMesh(
        core_axis_name="c", subcore_axis_name="s",
        num_cores=_NC, num_subcores=_NS),
    scratch_types=[
        pltpu.VMEM((_STAGE,), jnp.int32),
        pltpu.VMEM((_STAGE, _D), jnp.float32),
        pltpu.VMEM((_WIN, _D), jnp.float32),
        pltpu.VMEM_SHARED((2048, _D), jnp.float32),
        pltpu.SemaphoreType.DMA,
        pltpu.SemaphoreType.DMA,
    ],
)(_rpe_body)


def kernel(seq_len, rel_pos_emb):
    del seq_len
    return _rpe(rel_pos_emb)
